# baseline (speedup 1.0000x reference)
import functools

import jax
import jax.numpy as jnp
from jax import lax
from jax.experimental import pallas as pl
from jax.experimental.pallas import tpu as pltpu
from jax.experimental.pallas import tpu_sc as plsc

_NEG = 5
_EPS = 1e-5
_MAXN = 1.0 - _EPS

_NC = 2
_NS = 16
_NW = _NC * _NS
_C = 128
_L = 16


def _ceil_to(x, m):
    return ((x + m - 1) // m) * m


def _make_sc_dots(n_nodes, d, k_pos, k_neg):
    nsteps = d // _L
    ks = (k_pos, k_neg, k_pos, k_neg)
    mesh = plsc.VectorSubcoreMesh(core_axis_name="c", subcore_axis_name="s")
    out_type = []
    for k in ks:
        out_type += [jax.ShapeDtypeStruct((_NW * k,), jnp.float32)] * 3

    @functools.partial(
        pl.kernel,
        mesh=mesh,
        out_type=out_type,
        compiler_params=pltpu.CompilerParams(use_tc_tiling_on_sc=False, needs_layout_passes=False),
        scratch_types=[
            pltpu.VMEM((k_neg,), jnp.int32),
            pltpu.VMEM((k_neg,), jnp.int32),
            pltpu.VMEM((_C, d), jnp.float32),
            pltpu.VMEM((_C, d), jnp.float32),
            pltpu.VMEM((k_neg,), jnp.float32),
            pltpu.VMEM((k_neg,), jnp.float32),
            pltpu.VMEM((k_neg,), jnp.float32),
            pltpu.SemaphoreType.DMA,
            pltpu.SemaphoreType.DMA,
        ],
    )
    def sc_dots(h_sim, h_corr, ps_s, ps_d, ns_s, ns_d, pc_s, pc_d, nc_s, nc_d,
                o0, o1, o2, o3, o4, o5, o6, o7, o8, o9, o10, o11,
                idx_u, idx_v, rows_u, rows_v, out_uu, out_vv, out_uv,
                sem_u, sem_v):
        wid = lax.axis_index("s") * _NC + lax.axis_index("c")
        outs = (o0, o1, o2, o3, o4, o5, o6, o7, o8, o9, o10, o11)
        tabs = (h_sim, h_sim, h_corr, h_corr)
        srcs = (ps_s, ns_s, pc_s, nc_s)
        dsts = (ps_d, ns_d, pc_d, nc_d)
        for s in range(4):
            k = ks[s]
            base = wid * k
            pltpu.sync_copy(srcs[s].at[pl.ds(base, k)], idx_u.at[pl.ds(0, k)])
            pltpu.sync_copy(dsts[s].at[pl.ds(base, k)], idx_v.at[pl.ds(0, k)])

            def chunk_body(c, carry, s=s):
                off = c * _C
                cu = pltpu.make_async_copy(
                    tabs[s].at[idx_u.at[pl.ds(off, _C)]], rows_u, sem_u)
                cv = pltpu.make_async_copy(
                    tabs[s].at[idx_v.at[pl.ds(off, _C)]], rows_v, sem_v)
                cu.start()
                cv.start()
                cu.wait()
                cv.wait()

                iota = jnp.arange(_L, dtype=jnp.int32)

                def group_body(g, gcarry):
                    evec = g * _L + iota
                    zero = jnp.zeros((_L,), jnp.float32)

                    def col_body(j, accs):
                        auu, avv, auv = accs
                        col = jnp.full((_L,), 0, jnp.int32) + j
                        u = plsc.load_gather(rows_u, [evec, col])
                        v = plsc.load_gather(rows_v, [evec, col])
                        return (auu + u * u, avv + v * v, auv + u * v)

                    auu, avv, auv = lax.fori_loop(
                        0, d, col_body, (zero, zero, zero), unroll=8)
                    out_uu[pl.ds(off + g * _L, _L)] = auu
                    out_vv[pl.ds(off + g * _L, _L)] = avv
                    out_uv[pl.ds(off + g * _L, _L)] = auv
                    return gcarry

                lax.fori_loop(0, _C // _L, group_body, 0)
                return carry

            lax.fori_loop(0, k // _C, chunk_body, 0)
            for j, ob in enumerate((out_uu, out_vv, out_uv)):
                pltpu.sync_copy(ob.at[pl.ds(0, k)],
                                outs[3 * s + j].at[pl.ds(base, k)])

    return sc_dots


def _sigmoid(x):
    return 1.0 / (1.0 + jnp.exp(-x))


def _post_body(suu_p, svv_p, suv_p, suu_n, svv_n, suv_n,
               cuu_p, cvv_p, cuv_p, cuu_n, cvv_n, cuv_n,
               loss_ref, mrr_ref):
    def sim_score(uu, vv, uv):
        nu = jnp.sqrt(uu)
        nv = jnp.sqrt(vv)
        su = jnp.where(nu > _MAXN, _MAXN / jnp.maximum(nu, _EPS), 1.0)
        sv = jnp.where(nv > _MAXN, _MAXN / jnp.maximum(nv, _EPS), 1.0)
        spu = su * su * uu
        spv = sv * sv * vv
        sq = spu + spv - 2.0 * (su * sv * uv)
        gamma = 1.0 + 2.0 * sq / jnp.maximum((1.0 - spu) * (1.0 - spv), 1e-12)
        g = jnp.maximum(gamma, 1.0 + 1e-7)
        dist = jnp.log(g + jnp.sqrt((g - 1.0) * (g + 1.0)))
        return dist * dist

    def cos_score(uu, vv, uv):
        return uv / jnp.maximum(jnp.sqrt(uu) * jnp.sqrt(vv), 1e-8)

    sp = sim_score(suu_p[...], svv_p[...], suv_p[...])
    sn = sim_score(suu_n[...], svv_n[...], suv_n[...])
    pl_t = jnp.log(1.0 - _sigmoid(sp) + 1e-5)
    nl_t = jnp.log(_sigmoid(sn) + 1e-5)
    loss_sim = -jnp.mean(pl_t) - jnp.mean(nl_t)
    cnt = jnp.sum((sn <= sp).astype(jnp.float32), axis=0, keepdims=True)
    mrr_sim = jnp.mean(1.0 / (cnt + 1.0))

    cp = cos_score(cuu_p[...], cvv_p[...], cuv_p[...])
    cn = cos_score(cuu_n[...], cvv_n[...], cuv_n[...])
    loss_corr = -jnp.mean(cp) + jnp.mean(cn)
    cnt2 = jnp.sum((cn >= cp).astype(jnp.float32), axis=0, keepdims=True)
    mrr_corr = jnp.mean(1.0 / (cnt2 + 1.0))

    loss_ref[0, 0] = 0.5 * (loss_sim + loss_corr)
    mrr_ref[0, 0] = 0.5 * (mrr_sim + mrr_corr)


def kernel(h_sim, h_corr, pos_src_sim, pos_dst_sim, neg_src_sim, neg_dst_sim,
           pos_src_corr, pos_dst_corr, neg_src_corr, neg_dst_corr):
    n_nodes, d = h_sim.shape
    e_pos = pos_src_sim.shape[0]
    e_neg = neg_src_sim.shape[0]
    k_pos = _ceil_to(-(-e_pos // _NW), _C)
    k_neg = _ceil_to(-(-e_neg // _NW), _C)

    def pad(idx, k, e):
        return jnp.pad(idx.astype(jnp.int32), (0, _NW * k - e))

    sc = _make_sc_dots(n_nodes, d, k_pos, k_neg)
    outs = sc(h_sim, h_corr,
              pad(pos_src_sim, k_pos, e_pos), pad(pos_dst_sim, k_pos, e_pos),
              pad(neg_src_sim, k_neg, e_neg), pad(neg_dst_sim, k_neg, e_neg),
              pad(pos_src_corr, k_pos, e_pos), pad(pos_dst_corr, k_pos, e_pos),
              pad(neg_src_corr, k_neg, e_neg), pad(neg_dst_corr, k_neg, e_neg))

    def as_pos(a):
        return a[:e_pos][None, :]

    def as_neg(a):
        return a[:e_neg].reshape(e_neg // _NEG, _NEG).T

    post_in = [as_pos(outs[0]), as_pos(outs[1]), as_pos(outs[2]),
               as_neg(outs[3]), as_neg(outs[4]), as_neg(outs[5]),
               as_pos(outs[6]), as_pos(outs[7]), as_pos(outs[8]),
               as_neg(outs[9]), as_neg(outs[10]), as_neg(outs[11])]

    loss2, mrr2 = pl.pallas_call(
        _post_body,
        out_shape=[jax.ShapeDtypeStruct((1, 1), jnp.float32)] * 2,
        out_specs=[pl.BlockSpec(memory_space=pltpu.SMEM)] * 2,
    )(*post_in)
    return loss2[0, 0], mrr2[0, 0]


# speedup vs baseline: 2.5984x; 2.5984x over previous
import functools

import jax
import jax.numpy as jnp
from jax import lax
from jax.experimental import pallas as pl
from jax.experimental.pallas import tpu as pltpu
from jax.experimental.pallas import tpu_sc as plsc

_NEG = 5
_EPS = 1e-5
_MAXN = 1.0 - _EPS

_NC = 2
_NS = 16
_NW = _NC * _NS
_C = 128
_L = 16


def _ceil_to(x, m):
    return ((x + m - 1) // m) * m


def _make_sc_dots(n_nodes, d, k_pos, k_neg):
    nsteps = d // _L
    ks = (k_pos, k_neg, k_pos, k_neg)
    mesh = plsc.VectorSubcoreMesh(core_axis_name="c", subcore_axis_name="s")
    out_type = []
    for k in ks:
        out_type += [jax.ShapeDtypeStruct((_NW * k,), jnp.float32)] * 3

    @functools.partial(
        pl.kernel,
        mesh=mesh,
        out_type=out_type,
        compiler_params=pltpu.CompilerParams(use_tc_tiling_on_sc=False, needs_layout_passes=False),
        scratch_types=[
            pltpu.VMEM((k_neg,), jnp.int32),
            pltpu.VMEM((k_neg,), jnp.int32),
            pltpu.VMEM((_C, d), jnp.float32),
            pltpu.VMEM((_C, d), jnp.float32),
            pltpu.VMEM((k_neg,), jnp.float32),
            pltpu.VMEM((k_neg,), jnp.float32),
            pltpu.VMEM((k_neg,), jnp.float32),
            pltpu.SemaphoreType.DMA,
            pltpu.SemaphoreType.DMA,
        ],
    )
    def sc_dots(h_sim, h_corr, ps_s, ps_d, ns_s, ns_d, pc_s, pc_d, nc_s, nc_d,
                o0, o1, o2, o3, o4, o5, o6, o7, o8, o9, o10, o11,
                idx_u, idx_v, rows_u, rows_v, out_uu, out_vv, out_uv,
                sem_u, sem_v):
        wid = lax.axis_index("s") * _NC + lax.axis_index("c")
        outs = (o0, o1, o2, o3, o4, o5, o6, o7, o8, o9, o10, o11)
        tabs = (h_sim, h_sim, h_corr, h_corr)
        srcs = (ps_s, ns_s, pc_s, nc_s)
        dsts = (ps_d, ns_d, pc_d, nc_d)
        for s in range(4):
            k = ks[s]
            base = wid * k
            pltpu.sync_copy(srcs[s].at[pl.ds(base, k)], idx_u.at[pl.ds(0, k)])
            pltpu.sync_copy(dsts[s].at[pl.ds(base, k)], idx_v.at[pl.ds(0, k)])

            def chunk_body(c, carry, s=s):
                off = c * _C
                cu = pltpu.make_async_copy(
                    tabs[s].at[idx_u.at[pl.ds(off, _C)]], rows_u, sem_u)
                cv = pltpu.make_async_copy(
                    tabs[s].at[idx_v.at[pl.ds(off, _C)]], rows_v, sem_v)
                cu.start()
                cv.start()
                cu.wait()
                cv.wait()

                iota = jnp.arange(_L, dtype=jnp.int32)

                def group_body(g, gcarry):
                    evec = g * _L + iota
                    zero = jnp.zeros((_L,), jnp.float32)

                    def col_body(j, accs):
                        auu, avv, auv = accs
                        col = (iota + j) & (d - 1)
                        u = plsc.load_gather(rows_u, [evec, col])
                        v = plsc.load_gather(rows_v, [evec, col])
                        return (auu + u * u, avv + v * v, auv + u * v)

                    auu, avv, auv = lax.fori_loop(
                        0, d, col_body, (zero, zero, zero), unroll=8)
                    out_uu[pl.ds(off + g * _L, _L)] = auu
                    out_vv[pl.ds(off + g * _L, _L)] = avv
                    out_uv[pl.ds(off + g * _L, _L)] = auv
                    return gcarry

                lax.fori_loop(0, _C // _L, group_body, 0)
                return carry

            lax.fori_loop(0, k // _C, chunk_body, 0)
            for j, ob in enumerate((out_uu, out_vv, out_uv)):
                pltpu.sync_copy(ob.at[pl.ds(0, k)],
                                outs[3 * s + j].at[pl.ds(base, k)])

    return sc_dots


def _sigmoid(x):
    return 1.0 / (1.0 + jnp.exp(-x))


def _post_body(suu_p, svv_p, suv_p, suu_n, svv_n, suv_n,
               cuu_p, cvv_p, cuv_p, cuu_n, cvv_n, cuv_n,
               loss_ref, mrr_ref):
    def sim_score(uu, vv, uv):
        nu = jnp.sqrt(uu)
        nv = jnp.sqrt(vv)
        su = jnp.where(nu > _MAXN, _MAXN / jnp.maximum(nu, _EPS), 1.0)
        sv = jnp.where(nv > _MAXN, _MAXN / jnp.maximum(nv, _EPS), 1.0)
        spu = su * su * uu
        spv = sv * sv * vv
        sq = spu + spv - 2.0 * (su * sv * uv)
        gamma = 1.0 + 2.0 * sq / jnp.maximum((1.0 - spu) * (1.0 - spv), 1e-12)
        g = jnp.maximum(gamma, 1.0 + 1e-7)
        dist = jnp.log(g + jnp.sqrt((g - 1.0) * (g + 1.0)))
        return dist * dist

    def cos_score(uu, vv, uv):
        return uv / jnp.maximum(jnp.sqrt(uu) * jnp.sqrt(vv), 1e-8)

    sp = sim_score(suu_p[...], svv_p[...], suv_p[...])
    sn = sim_score(suu_n[...], svv_n[...], suv_n[...])
    pl_t = jnp.log(1.0 - _sigmoid(sp) + 1e-5)
    nl_t = jnp.log(_sigmoid(sn) + 1e-5)
    loss_sim = -jnp.mean(pl_t) - jnp.mean(nl_t)
    cnt = jnp.sum((sn <= sp).astype(jnp.float32), axis=0, keepdims=True)
    mrr_sim = jnp.mean(1.0 / (cnt + 1.0))

    cp = cos_score(cuu_p[...], cvv_p[...], cuv_p[...])
    cn = cos_score(cuu_n[...], cvv_n[...], cuv_n[...])
    loss_corr = -jnp.mean(cp) + jnp.mean(cn)
    cnt2 = jnp.sum((cn >= cp).astype(jnp.float32), axis=0, keepdims=True)
    mrr_corr = jnp.mean(1.0 / (cnt2 + 1.0))

    loss_ref[0, 0] = 0.5 * (loss_sim + loss_corr)
    mrr_ref[0, 0] = 0.5 * (mrr_sim + mrr_corr)


def kernel(h_sim, h_corr, pos_src_sim, pos_dst_sim, neg_src_sim, neg_dst_sim,
           pos_src_corr, pos_dst_corr, neg_src_corr, neg_dst_corr):
    n_nodes, d = h_sim.shape
    e_pos = pos_src_sim.shape[0]
    e_neg = neg_src_sim.shape[0]
    k_pos = _ceil_to(-(-e_pos // _NW), _C)
    k_neg = _ceil_to(-(-e_neg // _NW), _C)

    def pad(idx, k, e):
        return jnp.pad(idx.astype(jnp.int32), (0, _NW * k - e))

    sc = _make_sc_dots(n_nodes, d, k_pos, k_neg)
    outs = sc(h_sim, h_corr,
              pad(pos_src_sim, k_pos, e_pos), pad(pos_dst_sim, k_pos, e_pos),
              pad(neg_src_sim, k_neg, e_neg), pad(neg_dst_sim, k_neg, e_neg),
              pad(pos_src_corr, k_pos, e_pos), pad(pos_dst_corr, k_pos, e_pos),
              pad(neg_src_corr, k_neg, e_neg), pad(neg_dst_corr, k_neg, e_neg))

    def as_pos(a):
        return a[:e_pos][None, :]

    def as_neg(a):
        return a[:e_neg].reshape(e_neg // _NEG, _NEG).T

    post_in = [as_pos(outs[0]), as_pos(outs[1]), as_pos(outs[2]),
               as_neg(outs[3]), as_neg(outs[4]), as_neg(outs[5]),
               as_pos(outs[6]), as_pos(outs[7]), as_pos(outs[8]),
               as_neg(outs[9]), as_neg(outs[10]), as_neg(outs[11])]

    loss2, mrr2 = pl.pallas_call(
        _post_body,
        out_shape=[jax.ShapeDtypeStruct((1, 1), jnp.float32)] * 2,
        out_specs=[pl.BlockSpec(memory_space=pltpu.SMEM)] * 2,
    )(*post_in)
    return loss2[0, 0], mrr2[0, 0]


# speedup vs baseline: 3.0113x; 1.1589x over previous
import functools

import jax
import jax.numpy as jnp
from jax import lax
from jax.experimental import pallas as pl
from jax.experimental.pallas import tpu as pltpu
from jax.experimental.pallas import tpu_sc as plsc

_NEG = 5
_EPS = 1e-5
_MAXN = 1.0 - _EPS

_NC = 2
_NS = 16
_NW = _NC * _NS
_C = 64
_L = 16


def _ceil_to(x, m):
    return ((x + m - 1) // m) * m


def _make_sc_dots(n_nodes, d, k_pos, k_neg):
    nsteps = d // _L
    ks = (k_pos, k_neg, k_pos, k_neg)
    mesh = plsc.VectorSubcoreMesh(core_axis_name="c", subcore_axis_name="s")
    out_type = []
    for k in ks:
        out_type += [jax.ShapeDtypeStruct((_NW * k,), jnp.float32)] * 3

    @functools.partial(
        pl.kernel,
        mesh=mesh,
        out_type=out_type,
        compiler_params=pltpu.CompilerParams(use_tc_tiling_on_sc=False, needs_layout_passes=False),
        scratch_types=[
            pltpu.VMEM((k_neg,), jnp.int32),
            pltpu.VMEM((k_neg,), jnp.int32),
            pltpu.VMEM((_C, d), jnp.float32),
            pltpu.VMEM((_C, d), jnp.float32),
            pltpu.VMEM((_C, d), jnp.float32),
            pltpu.VMEM((_C, d), jnp.float32),
            pltpu.VMEM((k_neg,), jnp.float32),
            pltpu.VMEM((k_neg,), jnp.float32),
            pltpu.VMEM((k_neg,), jnp.float32),
            pltpu.SemaphoreType.DMA,
            pltpu.SemaphoreType.DMA,
            pltpu.SemaphoreType.DMA,
            pltpu.SemaphoreType.DMA,
        ],
    )
    def sc_dots(h_sim, h_corr, ps_s, ps_d, ns_s, ns_d, pc_s, pc_d, nc_s, nc_d,
                o0, o1, o2, o3, o4, o5, o6, o7, o8, o9, o10, o11,
                idx_u, idx_v, rows_u0, rows_v0, rows_u1, rows_v1,
                out_uu, out_vv, out_uv, sem_u0, sem_v0, sem_u1, sem_v1):
        wid = lax.axis_index("s") * _NC + lax.axis_index("c")
        outs = (o0, o1, o2, o3, o4, o5, o6, o7, o8, o9, o10, o11)
        tabs = (h_sim, h_sim, h_corr, h_corr)
        srcs = (ps_s, ns_s, pc_s, nc_s)
        dsts = (ps_d, ns_d, pc_d, nc_d)
        bufs = ((rows_u0, rows_v0, sem_u0, sem_v0),
                (rows_u1, rows_v1, sem_u1, sem_v1))
        iota = jnp.arange(_L, dtype=jnp.int32)

        def fire(tab, c, b):
            ru, rv, su, sv = bufs[b]
            pltpu.make_async_copy(
                tab.at[idx_u.at[pl.ds(c * _C, _C)]], ru, su).start()
            pltpu.make_async_copy(
                tab.at[idx_v.at[pl.ds(c * _C, _C)]], rv, sv).start()

        def consume(tab, c, b):
            ru, rv, su, sv = bufs[b]
            pltpu.make_async_copy(
                tab.at[idx_u.at[pl.ds(c * _C, _C)]], ru, su).wait()
            pltpu.make_async_copy(
                tab.at[idx_v.at[pl.ds(c * _C, _C)]], rv, sv).wait()
            off = c * _C

            def group_body(g, gcarry):
                evec = g * _L + iota
                zero = jnp.zeros((_L,), jnp.float32)

                def col_body(j2, accs):
                    auu0, avv0, auv0, auu1, avv1, auv1 = accs
                    c0 = (iota + 2 * j2) & (d - 1)
                    c1 = (iota + 2 * j2 + 1) & (d - 1)
                    u0 = plsc.load_gather(ru, [evec, c0])
                    v0 = plsc.load_gather(rv, [evec, c0])
                    u1 = plsc.load_gather(ru, [evec, c1])
                    v1 = plsc.load_gather(rv, [evec, c1])
                    return (auu0 + u0 * u0, avv0 + v0 * v0, auv0 + u0 * v0,
                            auu1 + u1 * u1, avv1 + v1 * v1, auv1 + u1 * v1)

                accs = lax.fori_loop(0, d // 2, col_body, (zero,) * 6,
                                     unroll=8)
                out_uu[pl.ds(off + g * _L, _L)] = accs[0] + accs[3]
                out_vv[pl.ds(off + g * _L, _L)] = accs[1] + accs[4]
                out_uv[pl.ds(off + g * _L, _L)] = accs[2] + accs[5]
                return gcarry

            lax.fori_loop(0, _C // _L, group_body, 0)

        for s in range(4):
            k = ks[s]
            nc = k // _C
            base = wid * k
            pltpu.sync_copy(srcs[s].at[pl.ds(base, k)], idx_u.at[pl.ds(0, k)])
            pltpu.sync_copy(dsts[s].at[pl.ds(base, k)], idx_v.at[pl.ds(0, k)])
            tab = tabs[s]
            fire(tab, 0, 0)

            def pair_body(p, carry, tab=tab, nc=nc):
                c0 = 2 * p
                fire(tab, c0 + 1, 1)
                consume(tab, c0, 0)

                @pl.when(c0 + 2 < nc)
                def _():
                    fire(tab, c0 + 2, 0)

                consume(tab, c0 + 1, 1)
                return carry

            lax.fori_loop(0, nc // 2, pair_body, 0)
            for j, ob in enumerate((out_uu, out_vv, out_uv)):
                pltpu.sync_copy(ob.at[pl.ds(0, k)],
                                outs[3 * s + j].at[pl.ds(base, k)])

    return sc_dots


def _sigmoid(x):
    return 1.0 / (1.0 + jnp.exp(-x))


def _post_body(suu_p, svv_p, suv_p, suu_n, svv_n, suv_n,
               cuu_p, cvv_p, cuv_p, cuu_n, cvv_n, cuv_n,
               loss_ref, mrr_ref):
    def sim_score(uu, vv, uv):
        nu = jnp.sqrt(uu)
        nv = jnp.sqrt(vv)
        su = jnp.where(nu > _MAXN, _MAXN / jnp.maximum(nu, _EPS), 1.0)
        sv = jnp.where(nv > _MAXN, _MAXN / jnp.maximum(nv, _EPS), 1.0)
        spu = su * su * uu
        spv = sv * sv * vv
        sq = spu + spv - 2.0 * (su * sv * uv)
        gamma = 1.0 + 2.0 * sq / jnp.maximum((1.0 - spu) * (1.0 - spv), 1e-12)
        g = jnp.maximum(gamma, 1.0 + 1e-7)
        dist = jnp.log(g + jnp.sqrt((g - 1.0) * (g + 1.0)))
        return dist * dist

    def cos_score(uu, vv, uv):
        return uv / jnp.maximum(jnp.sqrt(uu) * jnp.sqrt(vv), 1e-8)

    sp = sim_score(suu_p[...], svv_p[...], suv_p[...])
    sn = sim_score(suu_n[...], svv_n[...], suv_n[...])
    pl_t = jnp.log(1.0 - _sigmoid(sp) + 1e-5)
    nl_t = jnp.log(_sigmoid(sn) + 1e-5)
    loss_sim = -jnp.mean(pl_t) - jnp.mean(nl_t)
    cnt = jnp.sum((sn <= sp).astype(jnp.float32), axis=0, keepdims=True)
    mrr_sim = jnp.mean(1.0 / (cnt + 1.0))

    cp = cos_score(cuu_p[...], cvv_p[...], cuv_p[...])
    cn = cos_score(cuu_n[...], cvv_n[...], cuv_n[...])
    loss_corr = -jnp.mean(cp) + jnp.mean(cn)
    cnt2 = jnp.sum((cn >= cp).astype(jnp.float32), axis=0, keepdims=True)
    mrr_corr = jnp.mean(1.0 / (cnt2 + 1.0))

    loss_ref[0, 0] = 0.5 * (loss_sim + loss_corr)
    mrr_ref[0, 0] = 0.5 * (mrr_sim + mrr_corr)


def kernel(h_sim, h_corr, pos_src_sim, pos_dst_sim, neg_src_sim, neg_dst_sim,
           pos_src_corr, pos_dst_corr, neg_src_corr, neg_dst_corr):
    n_nodes, d = h_sim.shape
    e_pos = pos_src_sim.shape[0]
    e_neg = neg_src_sim.shape[0]
    k_pos = _ceil_to(-(-e_pos // _NW), 2 * _C)
    k_neg = _ceil_to(-(-e_neg // _NW), 2 * _C)

    def pad(idx, k, e):
        return jnp.pad(idx.astype(jnp.int32), (0, _NW * k - e))

    sc = _make_sc_dots(n_nodes, d, k_pos, k_neg)
    outs = sc(h_sim, h_corr,
              pad(pos_src_sim, k_pos, e_pos), pad(pos_dst_sim, k_pos, e_pos),
              pad(neg_src_sim, k_neg, e_neg), pad(neg_dst_sim, k_neg, e_neg),
              pad(pos_src_corr, k_pos, e_pos), pad(pos_dst_corr, k_pos, e_pos),
              pad(neg_src_corr, k_neg, e_neg), pad(neg_dst_corr, k_neg, e_neg))

    def as_pos(a):
        return a[:e_pos][None, :]

    def as_neg(a):
        return a[:e_neg].reshape(e_neg // _NEG, _NEG).T

    post_in = [as_pos(outs[0]), as_pos(outs[1]), as_pos(outs[2]),
               as_neg(outs[3]), as_neg(outs[4]), as_neg(outs[5]),
               as_pos(outs[6]), as_pos(outs[7]), as_pos(outs[8]),
               as_neg(outs[9]), as_neg(outs[10]), as_neg(outs[11])]

    loss2, mrr2 = pl.pallas_call(
        _post_body,
        out_shape=[jax.ShapeDtypeStruct((1, 1), jnp.float32)] * 2,
        out_specs=[pl.BlockSpec(memory_space=pltpu.SMEM)] * 2,
    )(*post_in)
    return loss2[0, 0], mrr2[0, 0]
